# Initial kernel scaffold; baseline (speedup 1.0000x reference)
import jax
import jax.numpy as jnp
from jax.experimental import pallas as pl


def kernel(x_email, ei_ue, ei_eu, lin_w, lin_b, emb_user, c1_ue_Wl, c1_ue_b, c1_ue_Wr, c1_eu_Wl, c1_eu_b, c1_eu_Wr, c2_ue_Wl, c2_ue_b, c2_ue_Wr, c2_eu_Wl, c2_eu_b, c2_eu_Wr, kan1_base, kan1_spline, kan2_base, kan2_spline):
    raise NotImplementedError("write your pallas kernel here")


# speedup vs baseline: 3.2959x; 3.2959x over previous
import functools

import jax
import jax.numpy as jnp
import numpy as np
from jax import lax
from jax.experimental import pallas as pl
from jax.experimental.pallas import tpu as pltpu
from jax.experimental.pallas import tpu_sc as plsc

N = 10000
H = 128
E = 320000
NC, NS = 2, 16
K = 128
CH = 80
GRP = 8
NGRP = CH // GRP
EPAD = NC * NS * CH * K
SEG = 10240
RPT = SEG // NS
GRID_SIZE = 5
SPLINE_ORDER = 3
NB = GRID_SIZE + SPLINE_ORDER
HID2 = 64
OUT = 2

_f32 = jnp.float32


_SC_MESH = plsc.VectorSubcoreMesh(
    core_axis_name="c", subcore_axis_name="s", num_cores=NC, num_subcores=NS)


@functools.partial(
    pl.kernel,
    out_type=(
        jax.ShapeDtypeStruct((NC * SEG, H), _f32),
        jax.ShapeDtypeStruct((NC * NS, SEG), _f32),
    ),
    mesh=_SC_MESH,
    compiler_params=pltpu.CompilerParams(needs_layout_passes=False),
    scratch_types=[
        pltpu.VMEM_SHARED((SEG, H), _f32),
        pltpu.VMEM((GRP, K), jnp.int32),
        pltpu.VMEM((GRP, K), jnp.int32),
        pltpu.VMEM((K, H), _f32),
        pltpu.VMEM((SEG,), _f32),
        pltpu.SemaphoreType.DMA,
    ],
)
def _segsum_sc(table, srcr, dstr, zrows, zcnt, out_s, out_c,
               acc, sbuf, dbuf, rows, cntv, sem0):
    c = lax.axis_index("c")
    s = lax.axis_index("s")
    w = c * NS + s
    r0 = s * RPT
    pltpu.sync_copy(zrows, rows)
    for t in range(RPT // K):
        pltpu.sync_copy(rows, acc.at[pl.ds(r0 + t * K, K)])
    pltpu.sync_copy(zcnt, cntv)
    plsc.subcore_barrier()

    ones16 = jnp.ones((16,), _f32)

    def body(g, carry):
        pltpu.sync_copy(srcr.at[pl.ds(w * CH + g * GRP, GRP)], sbuf)
        pltpu.sync_copy(dstr.at[pl.ds(w * CH + g * GRP, GRP)], dbuf)
        for k in range(GRP):
            d = pltpu.async_copy(table.at[sbuf.at[k]], rows, sem0)
            for i in range(K // 16):
                idx_v = dbuf[k, pl.ds(i * 16, 16)]
                plsc.addupdate_scatter(cntv, [idx_v], ones16)
            d.wait()
            pltpu.sync_copy(rows, acc.at[dbuf.at[k]], add=True)
        return carry

    lax.fori_loop(0, NGRP, body, 0)
    plsc.subcore_barrier()
    for t in range(RPT // K):
        pltpu.sync_copy(acc.at[pl.ds(r0 + t * K, K)], rows)
        pltpu.sync_copy(rows, out_s.at[pl.ds(c * SEG + r0 + t * K, K)])
    pltpu.sync_copy(cntv, out_c.at[w])


def _prep_edges(src, dst):
    pad = EPAD - E
    srcp = jnp.concatenate([src, jnp.zeros((pad,), jnp.int32)])
    dstp = jnp.concatenate([dst, jnp.full((pad,), N, jnp.int32)])
    return srcp.reshape(-1, K), dstp.reshape(-1, K)


def _segsum(table, srcr, dstr, consts):
    zrows, zcnt = consts
    out_s, out_c = _segsum_sc(table, srcr, dstr, zrows, zcnt)
    sa, sb = out_s[:N], out_s[SEG:SEG + N]
    cnt_t = out_c.T[:N]
    return sa, sb, cnt_t


_RB = 1000


def _dot(a, b):
    return jnp.dot(a, b, preferred_element_type=_f32)


def _linear_body(x_ref, wt_ref, b_ref, o_ref):
    o_ref[...] = _dot(x_ref[...], wt_ref[...]) + b_ref[...]


def _linear(x, wt, b):
    grid = N // _RB
    return pl.pallas_call(
        _linear_body,
        grid=(grid,),
        in_specs=[
            pl.BlockSpec((_RB, H), lambda i: (i, 0)),
            pl.BlockSpec((H, H), lambda i: (0, 0)),
            pl.BlockSpec((1, H), lambda i: (0, 0)),
        ],
        out_specs=pl.BlockSpec((_RB, H), lambda i: (i, 0)),
        out_shape=jax.ShapeDtypeStruct((N, H), _f32),
    )(x, wt, b)


def _combine_body(sa_ref, sb_ref, c_ref, xd_ref, wlt_ref, b_ref,
                  wrt_ref, o_ref):
    cntv = jnp.sum(c_ref[...], axis=1, keepdims=True)
    mean = (sa_ref[...] + sb_ref[...]) / jnp.maximum(cntv, 1.0)
    o_ref[...] = jnp.maximum(
        _dot(mean, wlt_ref[...]) + b_ref[...] + _dot(xd_ref[...], wrt_ref[...]),
        0.0)


def _combine(parts, x_dst, wl, b, wr):
    sa, sb, cnt_t = parts
    grid = N // _RB
    return pl.pallas_call(
        _combine_body,
        grid=(grid,),
        in_specs=[
            pl.BlockSpec((_RB, H), lambda i: (i, 0)),
            pl.BlockSpec((_RB, H), lambda i: (i, 0)),
            pl.BlockSpec((_RB, NC * NS), lambda i: (i, 0)),
            pl.BlockSpec((_RB, H), lambda i: (i, 0)),
            pl.BlockSpec((H, H), lambda i: (0, 0)),
            pl.BlockSpec((1, H), lambda i: (0, 0)),
            pl.BlockSpec((H, H), lambda i: (0, 0)),
        ],
        out_specs=pl.BlockSpec((_RB, H), lambda i: (i, 0)),
        out_shape=jax.ShapeDtypeStruct((N, H), _f32),
    )(sa, sb, cnt_t, x_dst, wl.T, b.reshape(1, H), wr.T)


def _knots():
    h = np.float32(2.0 / GRID_SIZE)
    g = np.arange(-SPLINE_ORDER, GRID_SIZE + SPLINE_ORDER + 1,
                  dtype=np.float32) * h - np.float32(1.0)
    return [np.float32(v) for v in g]


def _bspline_bases(x):
    g = _knots()
    nk = len(g)
    b = [jnp.logical_and(x >= g[j], x < g[j + 1]).astype(x.dtype)
         for j in range(nk - 1)]
    for k in range(1, SPLINE_ORDER + 1):
        nb = []
        for j in range(nk - 1 - k):
            left = (x - g[j]) / (g[j + k] - g[j])
            right = (g[j + k + 1] - x) / (g[j + k + 1] - g[j + 1])
            nb.append(left * b[j] + right * b[j + 1])
        b = nb
    return b


def _silu(x):
    return x * jax.nn.sigmoid(x)


def _kan_body(x_ref, b1t_ref, s1t_ref, b2t_ref, s2t_ref, o_ref):
    x = x_ref[...]
    h1 = _dot(_silu(x), b1t_ref[...])
    for j, base in enumerate(_bspline_bases(x)):
        h1 = h1 + _dot(base, s1t_ref[j])
    o = _dot(_silu(h1), b2t_ref[...])
    for j, base in enumerate(_bspline_bases(h1)):
        o = o + _dot(base, s2t_ref[j])
    o_ref[...] = o


def _kan(x, b1t, s1t, b2t, s2t):
    grid = N // _RB
    return pl.pallas_call(
        _kan_body,
        grid=(grid,),
        in_specs=[
            pl.BlockSpec((_RB, H), lambda i: (i, 0)),
            pl.BlockSpec((H, HID2), lambda i: (0, 0)),
            pl.BlockSpec((NB, H, HID2), lambda i: (0, 0, 0)),
            pl.BlockSpec((HID2, H), lambda i: (0, 0)),
            pl.BlockSpec((NB, HID2, H), lambda i: (0, 0, 0)),
        ],
        out_specs=pl.BlockSpec((_RB, H), lambda i: (i, 0)),
        out_shape=jax.ShapeDtypeStruct((N, H), _f32),
    )(x, b1t, s1t, b2t, s2t)


def kernel(x_email, ei_ue, ei_eu, lin_w, lin_b, emb_user,
           c1_ue_Wl, c1_ue_b, c1_ue_Wr, c1_eu_Wl, c1_eu_b, c1_eu_Wr,
           c2_ue_Wl, c2_ue_b, c2_ue_Wr, c2_eu_Wl, c2_eu_b, c2_eu_Wr,
           kan1_base, kan1_spline, kan2_base, kan2_spline):
    consts = (jnp.zeros((K, H), _f32), jnp.zeros((SEG,), _f32))
    sr_ue, dr_ue = _prep_edges(ei_ue[0], ei_ue[1])
    sr_eu, dr_eu = _prep_edges(ei_eu[0], ei_eu[1])

    x_e = _linear(x_email, lin_w.T, lin_b.reshape(1, H))
    x_u = emb_user

    p1e = _segsum(x_u, sr_ue, dr_ue, consts)
    p1u = _segsum(x_e, sr_eu, dr_eu, consts)
    e1 = _combine(p1e, x_e, c1_ue_Wl, c1_ue_b, c1_ue_Wr)
    u1 = _combine(p1u, x_u, c1_eu_Wl, c1_eu_b, c1_eu_Wr)
    p2e = _segsum(u1, sr_ue, dr_ue, consts)
    e2 = _combine(p2e, e1, c2_ue_Wl, c2_ue_b, c2_ue_Wr)

    b1t = kan1_base.T
    s1t = jnp.transpose(kan1_spline, (2, 1, 0))
    b2t = jnp.pad(kan2_base.T, ((0, 0), (0, H - OUT)))
    s2t = jnp.pad(jnp.transpose(kan2_spline, (2, 1, 0)),
                  ((0, 0), (0, 0), (0, H - OUT)))
    out = _kan(e2, b1t, s1t, b2t, s2t)
    return out[:, :OUT]


# speedup vs baseline: 3.6385x; 1.1040x over previous
import functools

import jax
import jax.numpy as jnp
import numpy as np
from jax import lax
from jax.experimental import pallas as pl
from jax.experimental.pallas import tpu as pltpu
from jax.experimental.pallas import tpu_sc as plsc

N = 10000
H = 128
E = 320000
NC, NS = 2, 16
K = 128
CH = 80
GRP = 8
NGRP = CH // GRP
EPAD = NC * NS * CH * K
SEG = 10240
RPT = SEG // NS
GRID_SIZE = 5
SPLINE_ORDER = 3
NB = GRID_SIZE + SPLINE_ORDER
HID2 = 64
OUT = 2

_f32 = jnp.float32


_SC_MESH = plsc.VectorSubcoreMesh(
    core_axis_name="c", subcore_axis_name="s", num_cores=NC, num_subcores=NS)


@functools.partial(
    pl.kernel,
    out_type=(
        jax.ShapeDtypeStruct((NC * SEG, H), _f32),
        jax.ShapeDtypeStruct((NC * NS, SEG), _f32),
    ),
    mesh=_SC_MESH,
    compiler_params=pltpu.CompilerParams(needs_layout_passes=False),
    scratch_types=[
        pltpu.VMEM_SHARED((SEG, H), _f32),
        pltpu.VMEM((GRP, K), jnp.int32),
        pltpu.VMEM((GRP, K), jnp.int32),
        pltpu.VMEM((2, K, H), _f32),
        pltpu.VMEM((SEG,), _f32),
        pltpu.SemaphoreType.DMA,
        pltpu.SemaphoreType.DMA,
    ],
)
def _segsum_sc(table, srcr, dstr, zrows, zcnt, out_s, out_c,
               acc, sbuf, dbuf, rows, cntv, sem0, sem1):
    c = lax.axis_index("c")
    s = lax.axis_index("s")
    w = c * NS + s
    r0 = s * RPT
    pltpu.sync_copy(zrows, rows.at[0])
    for t in range(RPT // K):
        pltpu.sync_copy(rows.at[0], acc.at[pl.ds(r0 + t * K, K)])
    pltpu.sync_copy(zcnt, cntv)
    plsc.subcore_barrier()

    ones16 = jnp.ones((16,), _f32)
    sems = (sem0, sem1)

    def body(g, carry):
        pltpu.sync_copy(srcr.at[pl.ds(w * CH + g * GRP, GRP)], sbuf)
        pltpu.sync_copy(dstr.at[pl.ds(w * CH + g * GRP, GRP)], dbuf)
        ds = [pltpu.async_copy(table.at[sbuf.at[0]], rows.at[0], sem0), None]
        for k in range(GRP):
            b = k & 1
            if k + 1 < GRP:
                ds[1 - b] = pltpu.async_copy(table.at[sbuf.at[k + 1]],
                                             rows.at[1 - b], sems[1 - b])
            for i in range(K // 16):
                idx_v = dbuf[k, pl.ds(i * 16, 16)]
                plsc.addupdate_scatter(cntv, [idx_v], ones16)
            ds[b].wait()
            pltpu.sync_copy(rows.at[b], acc.at[dbuf.at[k]], add=True)
        return carry

    lax.fori_loop(0, NGRP, body, 0)
    plsc.subcore_barrier()
    for t in range(RPT // K):
        pltpu.sync_copy(acc.at[pl.ds(r0 + t * K, K)], rows.at[0])
        pltpu.sync_copy(rows.at[0], out_s.at[pl.ds(c * SEG + r0 + t * K, K)])
    pltpu.sync_copy(cntv, out_c.at[w])


def _prep_edges(src, dst):
    pad = EPAD - E
    srcp = jnp.concatenate([src, jnp.zeros((pad,), jnp.int32)])
    dstp = jnp.concatenate([dst, jnp.full((pad,), N, jnp.int32)])
    return srcp.reshape(-1, K), dstp.reshape(-1, K)


def _segsum(table, srcr, dstr, consts):
    zrows, zcnt = consts
    out_s, out_c = _segsum_sc(table, srcr, dstr, zrows, zcnt)
    sa, sb = out_s[:N], out_s[SEG:SEG + N]
    cnt_t = out_c.T[:N]
    return sa, sb, cnt_t


_RB = 1000


def _dot(a, b):
    return jnp.dot(a, b, preferred_element_type=_f32)


def _linear_body(x_ref, wt_ref, b_ref, o_ref):
    o_ref[...] = _dot(x_ref[...], wt_ref[...]) + b_ref[...]


def _linear(x, wt, b):
    grid = N // _RB
    return pl.pallas_call(
        _linear_body,
        grid=(grid,),
        in_specs=[
            pl.BlockSpec((_RB, H), lambda i: (i, 0)),
            pl.BlockSpec((H, H), lambda i: (0, 0)),
            pl.BlockSpec((1, H), lambda i: (0, 0)),
        ],
        out_specs=pl.BlockSpec((_RB, H), lambda i: (i, 0)),
        out_shape=jax.ShapeDtypeStruct((N, H), _f32),
    )(x, wt, b)


def _combine_body(sa_ref, sb_ref, c_ref, xd_ref, wlt_ref, b_ref,
                  wrt_ref, o_ref):
    cntv = jnp.sum(c_ref[...], axis=1, keepdims=True)
    mean = (sa_ref[...] + sb_ref[...]) / jnp.maximum(cntv, 1.0)
    o_ref[...] = jnp.maximum(
        _dot(mean, wlt_ref[...]) + b_ref[...] + _dot(xd_ref[...], wrt_ref[...]),
        0.0)


def _combine(parts, x_dst, wl, b, wr):
    sa, sb, cnt_t = parts
    grid = N // _RB
    return pl.pallas_call(
        _combine_body,
        grid=(grid,),
        in_specs=[
            pl.BlockSpec((_RB, H), lambda i: (i, 0)),
            pl.BlockSpec((_RB, H), lambda i: (i, 0)),
            pl.BlockSpec((_RB, NC * NS), lambda i: (i, 0)),
            pl.BlockSpec((_RB, H), lambda i: (i, 0)),
            pl.BlockSpec((H, H), lambda i: (0, 0)),
            pl.BlockSpec((1, H), lambda i: (0, 0)),
            pl.BlockSpec((H, H), lambda i: (0, 0)),
        ],
        out_specs=pl.BlockSpec((_RB, H), lambda i: (i, 0)),
        out_shape=jax.ShapeDtypeStruct((N, H), _f32),
    )(sa, sb, cnt_t, x_dst, wl.T, b.reshape(1, H), wr.T)


def _knots():
    h = np.float32(2.0 / GRID_SIZE)
    g = np.arange(-SPLINE_ORDER, GRID_SIZE + SPLINE_ORDER + 1,
                  dtype=np.float32) * h - np.float32(1.0)
    return [np.float32(v) for v in g]


def _bspline_bases(x):
    g = _knots()
    nk = len(g)
    b = [jnp.logical_and(x >= g[j], x < g[j + 1]).astype(x.dtype)
         for j in range(nk - 1)]
    for k in range(1, SPLINE_ORDER + 1):
        nb = []
        for j in range(nk - 1 - k):
            left = (x - g[j]) / (g[j + k] - g[j])
            right = (g[j + k + 1] - x) / (g[j + k + 1] - g[j + 1])
            nb.append(left * b[j] + right * b[j + 1])
        b = nb
    return b


def _silu(x):
    return x * jax.nn.sigmoid(x)


def _kan_body(x_ref, b1t_ref, s1t_ref, b2t_ref, s2t_ref, o_ref):
    x = x_ref[...]
    h1 = _dot(_silu(x), b1t_ref[...])
    for j, base in enumerate(_bspline_bases(x)):
        h1 = h1 + _dot(base, s1t_ref[j])
    o = _dot(_silu(h1), b2t_ref[...])
    for j, base in enumerate(_bspline_bases(h1)):
        o = o + _dot(base, s2t_ref[j])
    o_ref[...] = o


def _kan(x, b1t, s1t, b2t, s2t):
    grid = N // _RB
    return pl.pallas_call(
        _kan_body,
        grid=(grid,),
        in_specs=[
            pl.BlockSpec((_RB, H), lambda i: (i, 0)),
            pl.BlockSpec((H, HID2), lambda i: (0, 0)),
            pl.BlockSpec((NB, H, HID2), lambda i: (0, 0, 0)),
            pl.BlockSpec((HID2, H), lambda i: (0, 0)),
            pl.BlockSpec((NB, HID2, H), lambda i: (0, 0, 0)),
        ],
        out_specs=pl.BlockSpec((_RB, H), lambda i: (i, 0)),
        out_shape=jax.ShapeDtypeStruct((N, H), _f32),
    )(x, b1t, s1t, b2t, s2t)


def kernel(x_email, ei_ue, ei_eu, lin_w, lin_b, emb_user,
           c1_ue_Wl, c1_ue_b, c1_ue_Wr, c1_eu_Wl, c1_eu_b, c1_eu_Wr,
           c2_ue_Wl, c2_ue_b, c2_ue_Wr, c2_eu_Wl, c2_eu_b, c2_eu_Wr,
           kan1_base, kan1_spline, kan2_base, kan2_spline):
    consts = (jnp.zeros((K, H), _f32), jnp.zeros((SEG,), _f32))
    sr_ue, dr_ue = _prep_edges(ei_ue[0], ei_ue[1])
    sr_eu, dr_eu = _prep_edges(ei_eu[0], ei_eu[1])

    x_e = _linear(x_email, lin_w.T, lin_b.reshape(1, H))
    x_u = emb_user

    p1e = _segsum(x_u, sr_ue, dr_ue, consts)
    p1u = _segsum(x_e, sr_eu, dr_eu, consts)
    e1 = _combine(p1e, x_e, c1_ue_Wl, c1_ue_b, c1_ue_Wr)
    u1 = _combine(p1u, x_u, c1_eu_Wl, c1_eu_b, c1_eu_Wr)
    p2e = _segsum(u1, sr_ue, dr_ue, consts)
    e2 = _combine(p2e, e1, c2_ue_Wl, c2_ue_b, c2_ue_Wr)

    b1t = kan1_base.T
    s1t = jnp.transpose(kan1_spline, (2, 1, 0))
    b2t = jnp.pad(kan2_base.T, ((0, 0), (0, H - OUT)))
    s2t = jnp.pad(jnp.transpose(kan2_spline, (2, 1, 0)),
                  ((0, 0), (0, 0), (0, H - OUT)))
    out = _kan(e2, b1t, s1t, b2t, s2t)
    return out[:, :OUT]


# speedup vs baseline: 3.8250x; 1.0513x over previous
import functools

import jax
import jax.numpy as jnp
import numpy as np
from jax import lax
from jax.experimental import pallas as pl
from jax.experimental.pallas import tpu as pltpu
from jax.experimental.pallas import tpu_sc as plsc

N = 10000
H = 128
E = 320000
NC, NS = 2, 16
K = 128
CH0 = 112
CH1 = 48
GRP = 8
EPAD = NS * (CH0 + CH1) * K
SEG = 10240
RPT = SEG // NS
GRID_SIZE = 5
SPLINE_ORDER = 3
NB = GRID_SIZE + SPLINE_ORDER
HID2 = 64
OUT = 2

_f32 = jnp.float32


_SC_MESH = plsc.VectorSubcoreMesh(
    core_axis_name="c", subcore_axis_name="s", num_cores=NC, num_subcores=NS)


@functools.partial(
    pl.kernel,
    out_type=(
        jax.ShapeDtypeStruct((NC * SEG, H), _f32),
        jax.ShapeDtypeStruct((NC * NS, SEG), _f32),
    ),
    mesh=_SC_MESH,
    compiler_params=pltpu.CompilerParams(needs_layout_passes=False),
    scratch_types=[
        pltpu.VMEM_SHARED((SEG, H), _f32),
        pltpu.VMEM((GRP, K), jnp.int32),
        pltpu.VMEM((GRP, K), jnp.int32),
        pltpu.VMEM((2, K, H), _f32),
        pltpu.VMEM((SEG,), _f32),
        pltpu.SemaphoreType.DMA,
        pltpu.SemaphoreType.DMA,
    ],
)
def _segsum_sc(table, srcr, dstr, zrows, zcnt, out_s, out_c,
               acc, sbuf, dbuf, rows, cntv, sem0, sem1):
    c = lax.axis_index("c")
    s = lax.axis_index("s")
    w = c * NS + s
    r0 = s * RPT
    base = jnp.where(c == 0, s * CH0, NS * CH0 + s * CH1)
    ngrp = jnp.where(c == 0, CH0 // GRP, CH1 // GRP)
    pltpu.sync_copy(zrows, rows.at[0])
    for t in range(RPT // K):
        pltpu.sync_copy(rows.at[0], acc.at[pl.ds(r0 + t * K, K)])
    pltpu.sync_copy(zcnt, cntv)
    plsc.subcore_barrier()

    ones16 = jnp.ones((16,), _f32)
    sems = (sem0, sem1)

    def body(g, carry):
        pltpu.sync_copy(srcr.at[pl.ds(base + g * GRP, GRP)], sbuf)
        pltpu.sync_copy(dstr.at[pl.ds(base + g * GRP, GRP)], dbuf)
        ds = [pltpu.async_copy(table.at[sbuf.at[0]], rows.at[0], sem0), None]
        for k in range(GRP):
            b = k & 1
            if k + 1 < GRP:
                ds[1 - b] = pltpu.async_copy(table.at[sbuf.at[k + 1]],
                                             rows.at[1 - b], sems[1 - b])
            for i in range(K // 16):
                idx_v = dbuf[k, pl.ds(i * 16, 16)]
                plsc.addupdate_scatter(cntv, [idx_v], ones16)
            ds[b].wait()
            pltpu.sync_copy(rows.at[b], acc.at[dbuf.at[k]], add=True)
        return carry

    lax.fori_loop(0, ngrp, body, 0)
    plsc.subcore_barrier()
    for t in range(RPT // K):
        pltpu.sync_copy(acc.at[pl.ds(r0 + t * K, K)], rows.at[0])
        pltpu.sync_copy(rows.at[0], out_s.at[pl.ds(c * SEG + r0 + t * K, K)])
    pltpu.sync_copy(cntv, out_c.at[w])


def _prep_edges(src, dst):
    pad = EPAD - E
    srcp = jnp.concatenate([src, jnp.zeros((pad,), jnp.int32)])
    dstp = jnp.concatenate([dst, jnp.full((pad,), N, jnp.int32)])
    return srcp.reshape(-1, K), dstp.reshape(-1, K)


def _segsum(table, srcr, dstr, consts):
    zrows, zcnt = consts
    out_s, out_c = _segsum_sc(table, srcr, dstr, zrows, zcnt)
    sa, sb = out_s[:N], out_s[SEG:SEG + N]
    cnt_t = out_c.T[:N]
    return sa, sb, cnt_t


_RB = 1000


def _dot(a, b):
    return jnp.dot(a, b, preferred_element_type=_f32)


def _linear_body(x_ref, wt_ref, b_ref, o_ref):
    o_ref[...] = _dot(x_ref[...], wt_ref[...]) + b_ref[...]


def _linear(x, wt, b):
    grid = N // _RB
    return pl.pallas_call(
        _linear_body,
        grid=(grid,),
        in_specs=[
            pl.BlockSpec((_RB, H), lambda i: (i, 0)),
            pl.BlockSpec((H, H), lambda i: (0, 0)),
            pl.BlockSpec((1, H), lambda i: (0, 0)),
        ],
        out_specs=pl.BlockSpec((_RB, H), lambda i: (i, 0)),
        out_shape=jax.ShapeDtypeStruct((N, H), _f32),
    )(x, wt, b)


def _combine_body(sa_ref, sb_ref, c_ref, xd_ref, wlt_ref, b_ref,
                  wrt_ref, o_ref):
    cntv = jnp.sum(c_ref[...], axis=1, keepdims=True)
    mean = (sa_ref[...] + sb_ref[...]) / jnp.maximum(cntv, 1.0)
    o_ref[...] = jnp.maximum(
        _dot(mean, wlt_ref[...]) + b_ref[...] + _dot(xd_ref[...], wrt_ref[...]),
        0.0)


def _combine(parts, x_dst, wl, b, wr):
    sa, sb, cnt_t = parts
    grid = N // _RB
    return pl.pallas_call(
        _combine_body,
        grid=(grid,),
        in_specs=[
            pl.BlockSpec((_RB, H), lambda i: (i, 0)),
            pl.BlockSpec((_RB, H), lambda i: (i, 0)),
            pl.BlockSpec((_RB, NC * NS), lambda i: (i, 0)),
            pl.BlockSpec((_RB, H), lambda i: (i, 0)),
            pl.BlockSpec((H, H), lambda i: (0, 0)),
            pl.BlockSpec((1, H), lambda i: (0, 0)),
            pl.BlockSpec((H, H), lambda i: (0, 0)),
        ],
        out_specs=pl.BlockSpec((_RB, H), lambda i: (i, 0)),
        out_shape=jax.ShapeDtypeStruct((N, H), _f32),
    )(sa, sb, cnt_t, x_dst, wl.T, b.reshape(1, H), wr.T)


def _knots():
    h = np.float32(2.0 / GRID_SIZE)
    g = np.arange(-SPLINE_ORDER, GRID_SIZE + SPLINE_ORDER + 1,
                  dtype=np.float32) * h - np.float32(1.0)
    return [np.float32(v) for v in g]


def _bspline_bases(x):
    g = _knots()
    nk = len(g)
    b = [jnp.logical_and(x >= g[j], x < g[j + 1]).astype(x.dtype)
         for j in range(nk - 1)]
    for k in range(1, SPLINE_ORDER + 1):
        nb = []
        for j in range(nk - 1 - k):
            left = (x - g[j]) / (g[j + k] - g[j])
            right = (g[j + k + 1] - x) / (g[j + k + 1] - g[j + 1])
            nb.append(left * b[j] + right * b[j + 1])
        b = nb
    return b


def _silu(x):
    return x * jax.nn.sigmoid(x)


def _kan_body(x_ref, b1t_ref, s1t_ref, b2t_ref, s2t_ref, o_ref):
    x = x_ref[...]
    h1 = _dot(_silu(x), b1t_ref[...])
    for j, base in enumerate(_bspline_bases(x)):
        h1 = h1 + _dot(base, s1t_ref[j])
    o = _dot(_silu(h1), b2t_ref[...])
    for j, base in enumerate(_bspline_bases(h1)):
        o = o + _dot(base, s2t_ref[j])
    o_ref[...] = o


def _kan(x, b1t, s1t, b2t, s2t):
    grid = N // _RB
    return pl.pallas_call(
        _kan_body,
        grid=(grid,),
        in_specs=[
            pl.BlockSpec((_RB, H), lambda i: (i, 0)),
            pl.BlockSpec((H, HID2), lambda i: (0, 0)),
            pl.BlockSpec((NB, H, HID2), lambda i: (0, 0, 0)),
            pl.BlockSpec((HID2, H), lambda i: (0, 0)),
            pl.BlockSpec((NB, HID2, H), lambda i: (0, 0, 0)),
        ],
        out_specs=pl.BlockSpec((_RB, H), lambda i: (i, 0)),
        out_shape=jax.ShapeDtypeStruct((N, H), _f32),
    )(x, b1t, s1t, b2t, s2t)


def kernel(x_email, ei_ue, ei_eu, lin_w, lin_b, emb_user,
           c1_ue_Wl, c1_ue_b, c1_ue_Wr, c1_eu_Wl, c1_eu_b, c1_eu_Wr,
           c2_ue_Wl, c2_ue_b, c2_ue_Wr, c2_eu_Wl, c2_eu_b, c2_eu_Wr,
           kan1_base, kan1_spline, kan2_base, kan2_spline):
    consts = (jnp.zeros((K, H), _f32), jnp.zeros((SEG,), _f32))
    sr_ue, dr_ue = _prep_edges(ei_ue[0], ei_ue[1])
    sr_eu, dr_eu = _prep_edges(ei_eu[0], ei_eu[1])

    x_e = _linear(x_email, lin_w.T, lin_b.reshape(1, H))
    x_u = emb_user

    p1e = _segsum(x_u, sr_ue, dr_ue, consts)
    p1u = _segsum(x_e, sr_eu, dr_eu, consts)
    e1 = _combine(p1e, x_e, c1_ue_Wl, c1_ue_b, c1_ue_Wr)
    u1 = _combine(p1u, x_u, c1_eu_Wl, c1_eu_b, c1_eu_Wr)
    p2e = _segsum(u1, sr_ue, dr_ue, consts)
    e2 = _combine(p2e, e1, c2_ue_Wl, c2_ue_b, c2_ue_Wr)

    b1t = kan1_base.T
    s1t = jnp.transpose(kan1_spline, (2, 1, 0))
    b2t = jnp.pad(kan2_base.T, ((0, 0), (0, H - OUT)))
    s2t = jnp.pad(jnp.transpose(kan2_spline, (2, 1, 0)),
                  ((0, 0), (0, 0), (0, H - OUT)))
    out = _kan(e2, b1t, s1t, b2t, s2t)
    return out[:, :OUT]


# speedup vs baseline: 3.8751x; 1.0131x over previous
import functools

import jax
import jax.numpy as jnp
import numpy as np
from jax import lax
from jax.experimental import pallas as pl
from jax.experimental.pallas import tpu as pltpu
from jax.experimental.pallas import tpu_sc as plsc

N = 10000
H = 128
E = 320000
NC, NS = 2, 16
K = 128
CH0 = 120
CH1 = 40
GRP = 8
EPAD = NS * (CH0 + CH1) * K
SEG = 10240
RPT = SEG // NS
GRID_SIZE = 5
SPLINE_ORDER = 3
NB = GRID_SIZE + SPLINE_ORDER
HID2 = 64
OUT = 2

_f32 = jnp.float32


_SC_MESH = plsc.VectorSubcoreMesh(
    core_axis_name="c", subcore_axis_name="s", num_cores=NC, num_subcores=NS)


@functools.partial(
    pl.kernel,
    out_type=(
        jax.ShapeDtypeStruct((NC * SEG, H), _f32),
        jax.ShapeDtypeStruct((NC * NS, SEG), _f32),
    ),
    mesh=_SC_MESH,
    compiler_params=pltpu.CompilerParams(needs_layout_passes=False),
    scratch_types=[
        pltpu.VMEM_SHARED((SEG, H), _f32),
        pltpu.VMEM((GRP, K), jnp.int32),
        pltpu.VMEM((GRP, K), jnp.int32),
        pltpu.VMEM((2, K, H), _f32),
        pltpu.VMEM((SEG,), _f32),
        pltpu.SemaphoreType.DMA,
        pltpu.SemaphoreType.DMA,
    ],
)
def _segsum_sc(table, srcr, dstr, zrows, zcnt, out_s, out_c,
               acc, sbuf, dbuf, rows, cntv, sem0, sem1):
    c = lax.axis_index("c")
    s = lax.axis_index("s")
    w = c * NS + s
    r0 = s * RPT
    base = jnp.where(c == 0, s * CH0, NS * CH0 + s * CH1)
    ngrp = jnp.where(c == 0, CH0 // GRP, CH1 // GRP)
    pltpu.sync_copy(zrows, rows.at[0])
    for t in range(RPT // K):
        pltpu.sync_copy(rows.at[0], acc.at[pl.ds(r0 + t * K, K)])
    pltpu.sync_copy(zcnt, cntv)
    plsc.subcore_barrier()

    ones16 = jnp.ones((16,), _f32)
    sems = (sem0, sem1)

    def body(g, carry):
        pltpu.sync_copy(srcr.at[pl.ds(base + g * GRP, GRP)], sbuf)
        pltpu.sync_copy(dstr.at[pl.ds(base + g * GRP, GRP)], dbuf)
        ds = [pltpu.async_copy(table.at[sbuf.at[0]], rows.at[0], sem0), None]
        for k in range(GRP):
            b = k & 1
            if k + 1 < GRP:
                ds[1 - b] = pltpu.async_copy(table.at[sbuf.at[k + 1]],
                                             rows.at[1 - b], sems[1 - b])
            for i in range(K // 16):
                idx_v = dbuf[k, pl.ds(i * 16, 16)]
                plsc.addupdate_scatter(cntv, [idx_v], ones16)
            ds[b].wait()
            pltpu.sync_copy(rows.at[b], acc.at[dbuf.at[k]], add=True)
        return carry

    lax.fori_loop(0, ngrp, body, 0)
    plsc.subcore_barrier()
    for t in range(RPT // K):
        pltpu.sync_copy(acc.at[pl.ds(r0 + t * K, K)], rows.at[0])
        pltpu.sync_copy(rows.at[0], out_s.at[pl.ds(c * SEG + r0 + t * K, K)])
    pltpu.sync_copy(cntv, out_c.at[w])


def _prep_edges(src, dst):
    pad = EPAD - E
    srcp = jnp.concatenate([src, jnp.zeros((pad,), jnp.int32)])
    dstp = jnp.concatenate([dst, jnp.full((pad,), N, jnp.int32)])
    return srcp.reshape(-1, K), dstp.reshape(-1, K)


def _segsum(table, srcr, dstr, consts):
    zrows, zcnt = consts
    out_s, out_c = _segsum_sc(table, srcr, dstr, zrows, zcnt)
    sa, sb = out_s[:N], out_s[SEG:SEG + N]
    cnt_t = out_c.T[:N]
    return sa, sb, cnt_t


_RB = 1000


def _dot(a, b):
    return jnp.dot(a, b, preferred_element_type=_f32)


def _linear_body(x_ref, wt_ref, b_ref, o_ref):
    o_ref[...] = _dot(x_ref[...], wt_ref[...]) + b_ref[...]


def _linear(x, wt, b):
    grid = N // _RB
    return pl.pallas_call(
        _linear_body,
        grid=(grid,),
        in_specs=[
            pl.BlockSpec((_RB, H), lambda i: (i, 0)),
            pl.BlockSpec((H, H), lambda i: (0, 0)),
            pl.BlockSpec((1, H), lambda i: (0, 0)),
        ],
        out_specs=pl.BlockSpec((_RB, H), lambda i: (i, 0)),
        out_shape=jax.ShapeDtypeStruct((N, H), _f32),
    )(x, wt, b)


def _combine_body(sa_ref, sb_ref, c_ref, xd_ref, wlt_ref, b_ref,
                  wrt_ref, o_ref):
    cntv = jnp.sum(c_ref[...], axis=1, keepdims=True)
    mean = (sa_ref[...] + sb_ref[...]) / jnp.maximum(cntv, 1.0)
    o_ref[...] = jnp.maximum(
        _dot(mean, wlt_ref[...]) + b_ref[...] + _dot(xd_ref[...], wrt_ref[...]),
        0.0)


def _combine(parts, x_dst, wl, b, wr):
    sa, sb, cnt_t = parts
    grid = N // _RB
    return pl.pallas_call(
        _combine_body,
        grid=(grid,),
        in_specs=[
            pl.BlockSpec((_RB, H), lambda i: (i, 0)),
            pl.BlockSpec((_RB, H), lambda i: (i, 0)),
            pl.BlockSpec((_RB, NC * NS), lambda i: (i, 0)),
            pl.BlockSpec((_RB, H), lambda i: (i, 0)),
            pl.BlockSpec((H, H), lambda i: (0, 0)),
            pl.BlockSpec((1, H), lambda i: (0, 0)),
            pl.BlockSpec((H, H), lambda i: (0, 0)),
        ],
        out_specs=pl.BlockSpec((_RB, H), lambda i: (i, 0)),
        out_shape=jax.ShapeDtypeStruct((N, H), _f32),
    )(sa, sb, cnt_t, x_dst, wl.T, b.reshape(1, H), wr.T)


def _knots():
    h = np.float32(2.0 / GRID_SIZE)
    g = np.arange(-SPLINE_ORDER, GRID_SIZE + SPLINE_ORDER + 1,
                  dtype=np.float32) * h - np.float32(1.0)
    return [np.float32(v) for v in g]


def _bspline_bases(x):
    g = _knots()
    nk = len(g)
    b = [jnp.logical_and(x >= g[j], x < g[j + 1]).astype(x.dtype)
         for j in range(nk - 1)]
    for k in range(1, SPLINE_ORDER + 1):
        nb = []
        for j in range(nk - 1 - k):
            left = (x - g[j]) / (g[j + k] - g[j])
            right = (g[j + k + 1] - x) / (g[j + k + 1] - g[j + 1])
            nb.append(left * b[j] + right * b[j + 1])
        b = nb
    return b


def _silu(x):
    return x * jax.nn.sigmoid(x)


def _kan_body(x_ref, b1t_ref, s1t_ref, b2t_ref, s2t_ref, o_ref):
    x = x_ref[...]
    h1 = _dot(_silu(x), b1t_ref[...])
    for j, base in enumerate(_bspline_bases(x)):
        h1 = h1 + _dot(base, s1t_ref[j])
    o = _dot(_silu(h1), b2t_ref[...])
    for j, base in enumerate(_bspline_bases(h1)):
        o = o + _dot(base, s2t_ref[j])
    o_ref[...] = o


def _kan(x, b1t, s1t, b2t, s2t):
    grid = N // _RB
    return pl.pallas_call(
        _kan_body,
        grid=(grid,),
        in_specs=[
            pl.BlockSpec((_RB, H), lambda i: (i, 0)),
            pl.BlockSpec((H, HID2), lambda i: (0, 0)),
            pl.BlockSpec((NB, H, HID2), lambda i: (0, 0, 0)),
            pl.BlockSpec((HID2, H), lambda i: (0, 0)),
            pl.BlockSpec((NB, HID2, H), lambda i: (0, 0, 0)),
        ],
        out_specs=pl.BlockSpec((_RB, H), lambda i: (i, 0)),
        out_shape=jax.ShapeDtypeStruct((N, H), _f32),
    )(x, b1t, s1t, b2t, s2t)


def kernel(x_email, ei_ue, ei_eu, lin_w, lin_b, emb_user,
           c1_ue_Wl, c1_ue_b, c1_ue_Wr, c1_eu_Wl, c1_eu_b, c1_eu_Wr,
           c2_ue_Wl, c2_ue_b, c2_ue_Wr, c2_eu_Wl, c2_eu_b, c2_eu_Wr,
           kan1_base, kan1_spline, kan2_base, kan2_spline):
    consts = (jnp.zeros((K, H), _f32), jnp.zeros((SEG,), _f32))
    sr_ue, dr_ue = _prep_edges(ei_ue[0], ei_ue[1])
    sr_eu, dr_eu = _prep_edges(ei_eu[0], ei_eu[1])

    x_e = _linear(x_email, lin_w.T, lin_b.reshape(1, H))
    x_u = emb_user

    p1e = _segsum(x_u, sr_ue, dr_ue, consts)
    p1u = _segsum(x_e, sr_eu, dr_eu, consts)
    e1 = _combine(p1e, x_e, c1_ue_Wl, c1_ue_b, c1_ue_Wr)
    u1 = _combine(p1u, x_u, c1_eu_Wl, c1_eu_b, c1_eu_Wr)
    p2e = _segsum(u1, sr_ue, dr_ue, consts)
    e2 = _combine(p2e, e1, c2_ue_Wl, c2_ue_b, c2_ue_Wr)

    b1t = kan1_base.T
    s1t = jnp.transpose(kan1_spline, (2, 1, 0))
    b2t = jnp.pad(kan2_base.T, ((0, 0), (0, H - OUT)))
    s2t = jnp.pad(jnp.transpose(kan2_spline, (2, 1, 0)),
                  ((0, 0), (0, 0), (0, H - OUT)))
    out = _kan(e2, b1t, s1t, b2t, s2t)
    return out[:, :OUT]
